# baseline (speedup 1.0000x reference)
import functools

import jax
import jax.numpy as jnp
from jax import lax
from jax.experimental import pallas as pl
from jax.experimental.pallas import tpu as pltpu
from jax.experimental.pallas import tpu_sc as plsc

N_NODES = 10000
D_FEAT = 128
N_EDGES = 320000

NC = 2
NS = 16
NW = NC * NS
LANES = 16

EDGES_PER_W = N_EDGES // NW
CHUNK = 80
N_CHUNKS = EDGES_PER_W // CHUNK
D_VECS = D_FEAT // LANES


def _sc_decode(z, edge_index):
    mesh = plsc.VectorSubcoreMesh(core_axis_name="c", subcore_axis_name="s")

    @functools.partial(
        pl.kernel,
        mesh=mesh,
        out_type=jax.ShapeDtypeStruct((N_EDGES,), jnp.float32),
        compiler_params=pltpu.CompilerParams(needs_layout_passes=False),
        scratch_types=[
            pltpu.VMEM((CHUNK,), jnp.int32),
            pltpu.VMEM((CHUNK,), jnp.int32),
            pltpu.VMEM((CHUNK,), jnp.int32),
            pltpu.VMEM((CHUNK,), jnp.int32),
            pltpu.VMEM((CHUNK, D_FEAT), jnp.float32),
            pltpu.VMEM((CHUNK, D_FEAT), jnp.float32),
            pltpu.VMEM((CHUNK, D_FEAT), jnp.float32),
            pltpu.VMEM((CHUNK, D_FEAT), jnp.float32),
            pltpu.VMEM((LANES, CHUNK), jnp.float32),
            pltpu.VMEM((CHUNK,), jnp.float32),
            pltpu.VMEM((CHUNK,), jnp.float32),
            pltpu.SemaphoreType.DMA,
            pltpu.SemaphoreType.DMA,
            pltpu.SemaphoreType.DMA,
        ],
    )
    def decode(z_hbm, ei_hbm, out_hbm,
               is0, it0, is1, it1, rs0, rt0, rs1, rt1, part, res0, res1,
               sem_i, sem_g0, sem_g1):
        wid = lax.axis_index("s") * NC + lax.axis_index("c")
        base = wid * EDGES_PER_W
        lane_iota = lax.iota(jnp.int32, LANES)

        def fetch_idx(off, i_s, i_t):
            a = pltpu.async_copy(ei_hbm.at[pl.ds(off, CHUNK)], i_s, sem_i)
            b = pltpu.async_copy(
                ei_hbm.at[pl.ds(N_EDGES + off, CHUNK)], i_t, sem_i)
            a.wait()
            b.wait()

        def start_gather(i_s, i_t, r_s, r_t, sem):
            pltpu.async_copy(z_hbm.at[i_s], r_s, sem)
            pltpu.async_copy(z_hbm.at[i_t], r_t, sem)

        def wait_gather(i_s, i_t, r_s, r_t, sem):
            pltpu.make_async_copy(z_hbm.at[i_s], r_s, sem).wait()
            pltpu.make_async_copy(z_hbm.at[i_t], r_t, sem).wait()

        def compute(off, r_s, r_t, res):
            @plsc.parallel_loop(0, CHUNK, unroll=8)
            def edge_body(e):
                acc = jnp.zeros((LANES,), jnp.float32)
                for k in range(D_VECS):
                    a = r_s[e, pl.ds(k * LANES, LANES)]
                    b = r_t[e, pl.ds(k * LANES, LANES)]
                    d = a - b
                    acc = acc + d * d
                plsc.store_scatter(
                    part, [lane_iota, jnp.full((LANES,), e, jnp.int32)], acc
                )

            for q in range(CHUNK // LANES):
                s = part[0, pl.ds(q * LANES, LANES)]
                for l in range(1, LANES):
                    s = s + part[l, pl.ds(q * LANES, LANES)]
                res[pl.ds(q * LANES, LANES)] = 1.0 / (1.0 + jnp.exp(s - 1.0))

            pltpu.sync_copy(res, out_hbm.at[pl.ds(off, CHUNK)])

        fetch_idx(base, is0, it0)
        start_gather(is0, it0, rs0, rt0, sem_g0)

        def pair_body(p, _):
            off0 = base + (2 * p) * CHUNK
            off1 = off0 + CHUNK
            off2 = off1 + CHUNK
            fetch_idx(off1, is1, it1)
            start_gather(is1, it1, rs1, rt1, sem_g1)
            wait_gather(is0, it0, rs0, rt0, sem_g0)
            compute(off0, rs0, rt0, res0)
            fetch_idx(off2, is0, it0)
            start_gather(is0, it0, rs0, rt0, sem_g0)
            wait_gather(is1, it1, rs1, rt1, sem_g1)
            compute(off1, rs1, rt1, res1)
            return 0

        lax.fori_loop(0, (N_CHUNKS - 1) // 2, pair_body, 0)

        wait_gather(is0, it0, rs0, rt0, sem_g0)
        compute(base + (N_CHUNKS - 1) * CHUNK, rs0, rt0, res0)

    return decode(z, edge_index)


def kernel(z, edge_index):
    return _sc_decode(z, edge_index.astype(jnp.int32).reshape(-1))


# speedup vs baseline: 1.4079x; 1.4079x over previous
import functools

import jax
import jax.numpy as jnp
from jax import lax
from jax.experimental import pallas as pl
from jax.experimental.pallas import tpu as pltpu
from jax.experimental.pallas import tpu_sc as plsc

N_NODES = 10000
D_FEAT = 128
N_EDGES = 320000

NC = 2
NS = 16
NW = NC * NS
LANES = 16

EDGES_PER_W = N_EDGES // NW
CHUNK = 80
N_CHUNKS = EDGES_PER_W // CHUNK
D_VECS = D_FEAT // LANES


def _sc_decode(z, edge_index):
    mesh = plsc.VectorSubcoreMesh(core_axis_name="c", subcore_axis_name="s")

    @functools.partial(
        pl.kernel,
        mesh=mesh,
        out_type=jax.ShapeDtypeStruct((N_EDGES,), jnp.float32),
        compiler_params=pltpu.CompilerParams(needs_layout_passes=False),
        scratch_types=[
            pltpu.VMEM((EDGES_PER_W,), jnp.int32),
            pltpu.VMEM((EDGES_PER_W,), jnp.int32),
            pltpu.VMEM((CHUNK, D_FEAT), jnp.float32),
            pltpu.VMEM((CHUNK, D_FEAT), jnp.float32),
            pltpu.VMEM((CHUNK, D_FEAT), jnp.float32),
            pltpu.VMEM((CHUNK, D_FEAT), jnp.float32),
            pltpu.VMEM((CHUNK,), jnp.float32),
            pltpu.VMEM((CHUNK,), jnp.float32),
            pltpu.SemaphoreType.DMA,
            pltpu.SemaphoreType.DMA,
            pltpu.SemaphoreType.DMA,
        ],
    )
    def decode(z_hbm, ei_hbm, out_hbm,
               idx_s, idx_t, rs0, rt0, rs1, rt1, res0, res1,
               sem_i, sem_g0, sem_g1):
        wid = lax.axis_index("s") * NC + lax.axis_index("c")
        base = wid * EDGES_PER_W
        last_lane = lax.iota(jnp.int32, LANES) == (LANES - 1)

        ci = pltpu.async_copy(ei_hbm.at[pl.ds(base, EDGES_PER_W)], idx_s, sem_i)
        cj = pltpu.async_copy(
            ei_hbm.at[pl.ds(N_EDGES + base, EDGES_PER_W)], idx_t, sem_i)
        ci.wait()
        cj.wait()

        def start_gather(c, r_s, r_t, sem):
            o = c * CHUNK
            pltpu.async_copy(z_hbm.at[idx_s.at[pl.ds(o, CHUNK)]], r_s, sem)
            pltpu.async_copy(z_hbm.at[idx_t.at[pl.ds(o, CHUNK)]], r_t, sem)

        def wait_gather(r_s, r_t, sem):
            pltpu.make_async_copy(z_hbm.at[idx_s.at[pl.ds(0, CHUNK)]], r_s,
                                  sem).wait()
            pltpu.make_async_copy(z_hbm.at[idx_t.at[pl.ds(0, CHUNK)]], r_t,
                                  sem).wait()

        def compute(c, r_s, r_t, res):
            @plsc.parallel_loop(0, CHUNK, unroll=8)
            def edge_body(e):
                acc = jnp.zeros((LANES,), jnp.float32)
                for k in range(D_VECS):
                    a = r_s[e, pl.ds(k * LANES, LANES)]
                    b = r_t[e, pl.ds(k * LANES, LANES)]
                    d = a - b
                    acc = acc + d * d
                tot = plsc.cumsum(acc)
                plsc.store_scatter(
                    res, [jnp.full((LANES,), e, jnp.int32)], tot,
                    mask=last_lane,
                )

            for q in range(CHUNK // LANES):
                v = res[pl.ds(q * LANES, LANES)]
                res[pl.ds(q * LANES, LANES)] = 1.0 / (1.0 + jnp.exp(v - 1.0))

            pltpu.sync_copy(res, out_hbm.at[pl.ds(base + c * CHUNK, CHUNK)])

        start_gather(0, rs0, rt0, sem_g0)

        def pair_body(p, _):
            c0 = 2 * p
            start_gather(c0 + 1, rs1, rt1, sem_g1)
            wait_gather(rs0, rt0, sem_g0)
            compute(c0, rs0, rt0, res0)
            start_gather(c0 + 2, rs0, rt0, sem_g0)
            wait_gather(rs1, rt1, sem_g1)
            compute(c0 + 1, rs1, rt1, res1)
            return 0

        lax.fori_loop(0, (N_CHUNKS - 1) // 2, pair_body, 0)

        wait_gather(rs0, rt0, sem_g0)
        compute(N_CHUNKS - 1, rs0, rt0, res0)

    return decode(z, edge_index)


def kernel(z, edge_index):
    return _sc_decode(z, edge_index.astype(jnp.int32).reshape(-1))
